# Initial kernel scaffold; baseline (speedup 1.0000x reference)
import jax
import jax.numpy as jnp
from jax.experimental import pallas as pl


def kernel(features, xyz, W1, b1, Wq, Wk, Wv, Wd1, bd1, Wd2, bd2, W2, b2):
    raise NotImplementedError("write your pallas kernel here")


# speedup vs baseline: 17.6637x; 17.6637x over previous
import functools

import jax
import jax.numpy as jnp
from jax import lax
from jax.experimental import pallas as pl
from jax.experimental.pallas import tpu as pltpu
from jax.experimental.pallas import tpu_sc as plsc

B, N, D, K = 4, 4096, 64, 16
R = 256
NT = N // R
XA = 8
D2 = 2 * D

NC, NS = 2, 16
NW = NC * NS
ROWS_TOTAL = B * N * K
RW = ROWS_TOTAL // NW
CH = 256
NCH = RW // CH


def _proj_body(f_ref, xyz_ref, wq_ref, bq_ref, wk_ref, bk_ref, wv_ref, bv_ref,
               wd1_ref, q_ref, kv_ref, e2_ref):
    f = f_ref[0]
    q_ref[0] = jnp.dot(f, wq_ref[...], preferred_element_type=jnp.float32) + bq_ref[...]
    k = jnp.dot(f, wk_ref[...], preferred_element_type=jnp.float32) + bk_ref[...]
    v = jnp.dot(f, wv_ref[...], preferred_element_type=jnp.float32) + bv_ref[...]
    kv_ref[...] = jnp.concatenate([k, v], axis=1)
    e = jnp.dot(xyz_ref[0], wd1_ref[...], preferred_element_type=jnp.float32)
    e2_ref[...] = jnp.concatenate([e, jnp.zeros((N, D), jnp.float32)], axis=1)


def _knn_body(lhs_ref, rhs_ref, sqc_ref, sqr_ref, idx_ref):
    dot = jnp.dot(lhs_ref[0], rhs_ref[0], preferred_element_type=jnp.float32)
    d = (sqc_ref[0] + sqr_ref[0]) - 2.0 * dot
    iota = lax.broadcasted_iota(jnp.int32, (R, N), 1)
    cols = []
    for _ in range(K):
        m = jnp.min(d, axis=1, keepdims=True)
        sel = jnp.min(jnp.where(d <= m, iota, N), axis=1, keepdims=True)
        cols.append(sel)
        d = jnp.where(iota == sel, jnp.inf, d)
    idx = jnp.concatenate(cols, axis=1)
    b = pl.program_id(0)
    idx_ref[0] = idx + N * b


def _attn_body(q_ref, kvg_ref, eg_ref, xyz_ref, f_ref,
               wd1_ref, bd1_ref, wd2_ref, bd2_ref, w2_ref, b2_ref, out_ref):
    ei = jnp.dot(xyz_ref[0], wd1_ref[...],
                 preferred_element_type=jnp.float32) + bd1_ref[...]
    e3 = eg_ref[:, :D].reshape(R, K, D)
    h = jnp.maximum(ei[:, None, :] - e3, 0.0)
    pos = jnp.dot(h.reshape(R * K, D), wd2_ref[...],
                  preferred_element_type=jnp.float32) + bd2_ref[...]
    kpe = (kvg_ref[:, :D] + pos).reshape(R, K, D)
    logits = jnp.sum(q_ref[0][:, None, :] * kpe, axis=2) * 0.125
    m = jnp.max(logits, axis=1, keepdims=True)
    e = jnp.exp(logits - m)
    a = e / jnp.sum(e, axis=1, keepdims=True)
    vpe = (kvg_ref[:, D:] + pos).reshape(R, K, D)
    o = jnp.sum(a[:, :, None] * vpe, axis=1)
    out_ref[0] = jnp.dot(o, w2_ref[...], preferred_element_type=jnp.float32) \
        + b2_ref[...] + f_ref[0]


def _sc_gather_body(idx_hbm, kv_hbm, e2_hbm,
                    kvg_hbm, eg_hbm,
                    idx_v, kvb, eb, s1, s2):
    wid = lax.axis_index("s") * NC + lax.axis_index("c")
    base = wid * RW
    pltpu.sync_copy(idx_hbm.at[pl.ds(base, RW)], idx_v)

    def chunk(j, carry):
        off = j * CH
        ids = idx_v.at[pl.ds(off, CH)]
        c1 = pltpu.async_copy(kv_hbm.at[ids], kvb, s1)
        c2 = pltpu.async_copy(e2_hbm.at[ids], eb, s2)
        c1.wait()
        c2.wait()
        pltpu.sync_copy(kvb, kvg_hbm.at[pl.ds(base + off, CH)])
        pltpu.sync_copy(eb, eg_hbm.at[pl.ds(base + off, CH)])
        return carry

    lax.fori_loop(0, NCH, chunk, 0)


@functools.cache
def _sc_gather():
    return pl.kernel(
        _sc_gather_body,
        out_type=(
            jax.ShapeDtypeStruct((ROWS_TOTAL, D2), jnp.float32),
            jax.ShapeDtypeStruct((ROWS_TOTAL, D2), jnp.float32),
        ),
        mesh=plsc.VectorSubcoreMesh(core_axis_name="c", subcore_axis_name="s",
                                    num_cores=NC, num_subcores=NS),
        scratch_types=[
            pltpu.VMEM((RW,), jnp.int32),
            pltpu.VMEM((CH, D2), jnp.float32),
            pltpu.VMEM((CH, D2), jnp.float32),
            pltpu.SemaphoreType.DMA,
            pltpu.SemaphoreType.DMA,
        ],
    )


def kernel(features, xyz, W1, b1, Wq, Wk, Wv, Wd1, bd1, Wd2, bd2, W2, b2):
    f32 = jnp.float32
    wq = W1 @ Wq
    wk = W1 @ Wk
    wv = W1 @ Wv
    bq = (b1 @ Wq)[None, :]
    bk = (b1 @ Wk)[None, :]
    bv = (b1 @ Wv)[None, :]
    wd1p = jnp.concatenate([Wd1, jnp.zeros((XA - 3, D), f32)], axis=0)

    sq = jnp.sum(xyz * xyz, axis=-1)
    sqc = sq[..., None]
    sqr = sq[:, None, :]
    xyzp = jnp.concatenate([xyz, jnp.zeros((B, N, XA - 3), f32)], axis=-1)
    rhs = jnp.transpose(xyzp, (0, 2, 1))

    q_full, kv_flat, e2_flat = pl.pallas_call(
        _proj_body,
        grid=(B,),
        in_specs=[
            pl.BlockSpec((1, N, D), lambda b: (b, 0, 0)),
            pl.BlockSpec((1, N, XA), lambda b: (b, 0, 0)),
            pl.BlockSpec((D, D), lambda b: (0, 0)),
            pl.BlockSpec((1, D), lambda b: (0, 0)),
            pl.BlockSpec((D, D), lambda b: (0, 0)),
            pl.BlockSpec((1, D), lambda b: (0, 0)),
            pl.BlockSpec((D, D), lambda b: (0, 0)),
            pl.BlockSpec((1, D), lambda b: (0, 0)),
            pl.BlockSpec((XA, D), lambda b: (0, 0)),
        ],
        out_specs=[
            pl.BlockSpec((1, N, D), lambda b: (b, 0, 0)),
            pl.BlockSpec((N, D2), lambda b: (b, 0)),
            pl.BlockSpec((N, D2), lambda b: (b, 0)),
        ],
        out_shape=[
            jax.ShapeDtypeStruct((B, N, D), f32),
            jax.ShapeDtypeStruct((B * N, D2), f32),
            jax.ShapeDtypeStruct((B * N, D2), f32),
        ],
    )(features, xyzp, wq, bq, wk, bk, wv, bv, wd1p)

    knn_idx = pl.pallas_call(
        _knn_body,
        grid=(B, NT),
        in_specs=[
            pl.BlockSpec((1, R, XA), lambda b, i: (b, i, 0)),
            pl.BlockSpec((1, XA, N), lambda b, i: (b, 0, 0)),
            pl.BlockSpec((1, R, 1), lambda b, i: (b, i, 0)),
            pl.BlockSpec((1, 1, N), lambda b, i: (b, 0, 0)),
        ],
        out_specs=pl.BlockSpec((1, R, K), lambda b, i: (b, i, 0)),
        out_shape=jax.ShapeDtypeStruct((B, N, K), jnp.int32),
    )(xyzp, rhs, sqc, sqr)

    idx_flat = knn_idx.reshape(ROWS_TOTAL)
    kvg, eg = _sc_gather()(idx_flat, kv_flat, e2_flat)

    res = pl.pallas_call(
        _attn_body,
        grid=(B, NT),
        in_specs=[
            pl.BlockSpec((1, R, D), lambda b, i: (b, i, 0)),
            pl.BlockSpec((R * K, D2), lambda b, i: (b * NT + i, 0)),
            pl.BlockSpec((R * K, D2), lambda b, i: (b * NT + i, 0)),
            pl.BlockSpec((1, R, XA), lambda b, i: (b, i, 0)),
            pl.BlockSpec((1, R, D), lambda b, i: (b, i, 0)),
            pl.BlockSpec((XA, D), lambda b, i: (0, 0)),
            pl.BlockSpec((1, D), lambda b, i: (0, 0)),
            pl.BlockSpec((D, D), lambda b, i: (0, 0)),
            pl.BlockSpec((1, D), lambda b, i: (0, 0)),
            pl.BlockSpec((D, D), lambda b, i: (0, 0)),
            pl.BlockSpec((1, D), lambda b, i: (0, 0)),
        ],
        out_specs=pl.BlockSpec((1, R, D), lambda b, i: (b, i, 0)),
        out_shape=jax.ShapeDtypeStruct((B, N, D), f32),
    )(q_full, kvg, eg, xyzp, features,
      wd1p, bd1[None, :], Wd2, bd2[None, :], W2, b2[None, :])

    return jnp.transpose(res, (0, 2, 1))
